# baseline (speedup 1.0000x reference)
import functools

import jax
import jax.numpy as jnp
from jax import lax
from jax.experimental import pallas as pl
from jax.experimental.pallas import tpu as pltpu
from jax.experimental.pallas import tpu_sc as plsc

LEVELS = 6
NW = 32
CHUNK = 8


def _pyramid_body(ft_ref, p_ref):
    x = jnp.transpose(ft_ref[0], (1, 0)).astype(jnp.bfloat16)
    t = x.shape[0]
    p_ref[0, 0] = x
    level = x
    d = 1
    for k in range(1, LEVELS):
        shifted = jnp.concatenate([level[d:, :], level[t - d:, :]], axis=0)
        level = jnp.maximum(level, shifted)
        p_ref[0, k] = level
        d *= 2


def _build_pyramid(feature, b, t, ch):
    return pl.pallas_call(
        _pyramid_body,
        grid=(2, b),
        in_specs=[pl.BlockSpec((1, ch, t), lambda h, bb: (bb, h, 0))],
        out_specs=pl.BlockSpec((1, LEVELS, t, ch),
                               lambda h, bb: (h, 0, 0, bb)),
        out_shape=jax.ShapeDtypeStruct((2, LEVELS, t, b * ch), jnp.bfloat16),
    )(feature)


def _gather_rows(p2d, idx, n_out, d):
    per_w = n_out // NW
    nchunk = per_w // CHUNK
    mesh = plsc.VectorSubcoreMesh(core_axis_name="c", subcore_axis_name="s")

    @functools.partial(
        pl.kernel,
        mesh=mesh,
        out_type=jax.ShapeDtypeStruct((n_out, d), jnp.int32),
        scratch_types=[
            pltpu.VMEM((per_w,), jnp.int32),
            pltpu.VMEM((CHUNK, d), jnp.int32),
            pltpu.VMEM((CHUNK, d), jnp.int32),
            pltpu.SemaphoreType.DMA,
            pltpu.SemaphoreType.DMA,
            pltpu.SemaphoreType.DMA,
            pltpu.SemaphoreType.DMA,
        ],
    )
    def k(p_hbm, i_hbm, g_hbm, i_v, r0_v, r1_v, si0, si1, so0, so1):
        wid = lax.axis_index("s") * 2 + lax.axis_index("c")
        base = wid * per_w
        pltpu.sync_copy(i_hbm.at[pl.ds(base, per_w)], i_v)

        bufs = (r0_v, r1_v)
        isems = (si0, si1)
        osems = (so0, so1)

        def gather(ci):
            return pltpu.async_copy(
                p_hbm.at[i_v.at[pl.ds(ci * CHUNK, CHUNK)]],
                bufs[ci % 2], isems[ci % 2])

        def write(ci):
            return pltpu.async_copy(
                bufs[ci % 2], g_hbm.at[pl.ds(base + ci * CHUNK, CHUNK)],
                osems[ci % 2])

        gs = {0: gather(0)}
        if nchunk > 1:
            gs[1] = gather(1)
        ws = {}
        for ci in range(nchunk):
            gs[ci].wait()
            ws[ci] = write(ci)
            if ci + 2 < nchunk:
                ws[ci].wait()
                gs[ci + 2] = gather(ci + 2)
        for ci in range(max(0, nchunk - 2), nchunk):
            ws[ci].wait()

    return k(p2d, idx)


def _epilogue_body(g_ref, o_ref):
    m = jnp.maximum(g_ref[0, 0], g_ref[1, 0])
    o_ref[0] = jnp.transpose(m, (1, 0)).astype(jnp.float32)


def _pair_max_transpose(g4, b, ch, nseg):
    return pl.pallas_call(
        _epilogue_body,
        grid=(2, b),
        in_specs=[pl.BlockSpec((2, 1, nseg, ch), lambda h, bb: (0, h, 0, bb))],
        out_specs=pl.BlockSpec((1, ch, nseg), lambda h, bb: (bb, h, 0)),
        out_shape=jax.ShapeDtypeStruct((b, 2 * ch, nseg), jnp.float32),
    )(g4)


def kernel(feature, segments, max_len):
    b, c2, t = feature.shape
    ch = c2 // 2
    nseg = segments.shape[1]

    max_val = jnp.asarray(max_len - 1, dtype=segments.dtype)
    seg = jnp.clip(segments[0], 0.0, max_val)
    lo = jnp.stack([jnp.floor(seg[:, 0]), jnp.floor(seg[:, 2])])
    hi = jnp.stack([jnp.ceil(seg[:, 1]), jnp.ceil(seg[:, 3])])
    lo = lo.astype(jnp.int32)
    hi = jnp.maximum(hi.astype(jnp.int32), lo + 1)
    w = hi - lo
    k = jnp.minimum(31 - lax.clz(w), LEVELS - 1)
    pw = jnp.left_shift(jnp.int32(1), k)
    hoff = (jnp.arange(2, dtype=jnp.int32) * (LEVELS * t))[:, None]
    idx1 = (hoff + k * t + lo).reshape(-1)
    idx2 = (hoff + k * t + (hi - pw)).reshape(-1)
    nrows = 2 * LEVELS * t
    idx1 = jnp.clip(idx1, 0, nrows - 1)
    idx2 = jnp.clip(idx2, 0, nrows - 1)

    pyr = _build_pyramid(feature, b, t, ch)
    d32 = b * ch // 2
    p2d = lax.bitcast_convert_type(
        pyr.reshape(nrows, d32, 2), jnp.int32)

    idx = jnp.concatenate([idx1, idx2])
    g = _gather_rows(p2d, idx, 4 * nseg, d32)

    g4 = lax.bitcast_convert_type(
        g.reshape(2, 2, nseg, d32), jnp.bfloat16)
    g4 = g4.reshape(2, 2, nseg, b * ch)
    return _pair_max_transpose(g4, b, ch, nseg)


# speedup vs baseline: 12.0313x; 12.0313x over previous
import functools

import jax
import jax.numpy as jnp
from jax import lax
from jax.experimental import pallas as pl
from jax.experimental.pallas import tpu as pltpu
from jax.experimental.pallas import tpu_sc as plsc

LEVELS = 6
NW = 32
CHUNK = 8


def _rnd_bf16_bits(u):
    return (u + jnp.uint32(0x7FFF) + ((u >> 16) & jnp.uint32(1))) >> 16


def _pack_pair(lo, hi):
    ul = _rnd_bf16_bits(jax.lax.bitcast_convert_type(lo, jnp.uint32))
    uh = _rnd_bf16_bits(jax.lax.bitcast_convert_type(hi, jnp.uint32))
    return jax.lax.bitcast_convert_type((uh << 16) | ul, jnp.int32)


def _unpack_pair(w):
    u = jax.lax.bitcast_convert_type(w, jnp.uint32)
    lo = jax.lax.bitcast_convert_type(u << 16, jnp.float32)
    hi = jax.lax.bitcast_convert_type(u & jnp.uint32(0xFFFF0000), jnp.float32)
    return lo, hi


def _pyramid_body(ft_ref, p_ref):
    x = jnp.transpose(ft_ref[0], (1, 0))
    t, c = x.shape
    level = x
    d = 1
    for k in range(LEVELS):
        if k > 0:
            shifted = jnp.concatenate([level[d:, :], level[t - d:, :]],
                                      axis=0)
            level = jnp.maximum(level, shifted)
            d *= 2
        p_ref[0, k] = _pack_pair(level[:, : c // 2], level[:, c // 2:])


def _build_pyramid(feature, b, t, ch):
    return pl.pallas_call(
        _pyramid_body,
        grid=(2, b),
        in_specs=[pl.BlockSpec((1, ch, t), lambda h, bb: (bb, h, 0))],
        out_specs=pl.BlockSpec((1, LEVELS, t, ch // 2),
                               lambda h, bb: (h, 0, 0, bb)),
        out_shape=jax.ShapeDtypeStruct((2, LEVELS, t, b * ch // 2),
                                       jnp.int32),
    )(feature)


def _gather_rows(p2d, idx, n_out, d):
    per_w = n_out // NW
    nchunk = per_w // CHUNK
    mesh = plsc.VectorSubcoreMesh(core_axis_name="c", subcore_axis_name="s")

    @functools.partial(
        pl.kernel,
        mesh=mesh,
        out_type=jax.ShapeDtypeStruct((n_out, d), jnp.int32),
        scratch_types=[
            pltpu.VMEM((per_w,), jnp.int32),
            pltpu.VMEM((CHUNK, d), jnp.int32),
            pltpu.VMEM((CHUNK, d), jnp.int32),
            pltpu.SemaphoreType.DMA,
            pltpu.SemaphoreType.DMA,
            pltpu.SemaphoreType.DMA,
            pltpu.SemaphoreType.DMA,
        ],
    )
    def k(p_hbm, i_hbm, g_hbm, i_v, r0_v, r1_v, si0, si1, so0, so1):
        wid = lax.axis_index("s") * 2 + lax.axis_index("c")
        base = wid * per_w
        pltpu.sync_copy(i_hbm.at[pl.ds(base, per_w)], i_v)

        bufs = (r0_v, r1_v)
        isems = (si0, si1)
        osems = (so0, so1)

        def gather(ci):
            return pltpu.async_copy(
                p_hbm.at[i_v.at[pl.ds(ci * CHUNK, CHUNK)]],
                bufs[ci % 2], isems[ci % 2])

        def write(ci):
            return pltpu.async_copy(
                bufs[ci % 2], g_hbm.at[pl.ds(base + ci * CHUNK, CHUNK)],
                osems[ci % 2])

        gs = {0: gather(0)}
        if nchunk > 1:
            gs[1] = gather(1)
        ws = {}
        for ci in range(nchunk):
            gs[ci].wait()
            ws[ci] = write(ci)
            if ci + 2 < nchunk:
                ws[ci].wait()
                gs[ci + 2] = gather(ci + 2)
        for ci in range(max(0, nchunk - 2), nchunk):
            ws[ci].wait()

    return k(p2d, idx)


def _epilogue_body(g_ref, o_ref):
    a_lo, a_hi = _unpack_pair(g_ref[0, 0])
    b_lo, b_hi = _unpack_pair(g_ref[1, 0])
    m = jnp.concatenate([jnp.maximum(a_lo, b_lo),
                         jnp.maximum(a_hi, b_hi)], axis=1)
    o_ref[0] = jnp.transpose(m, (1, 0))


def _pair_max_transpose(g4, b, ch, nseg):
    return pl.pallas_call(
        _epilogue_body,
        grid=(2, b),
        in_specs=[pl.BlockSpec((2, 1, nseg, ch // 2),
                               lambda h, bb: (0, h, 0, bb))],
        out_specs=pl.BlockSpec((1, ch, nseg), lambda h, bb: (bb, h, 0)),
        out_shape=jax.ShapeDtypeStruct((b, 2 * ch, nseg), jnp.float32),
    )(g4)


def kernel(feature, segments, max_len):
    b, c2, t = feature.shape
    ch = c2 // 2
    nseg = segments.shape[1]

    max_val = jnp.asarray(max_len - 1, dtype=segments.dtype)
    seg = jnp.clip(segments[0], 0.0, max_val)
    lo = jnp.stack([jnp.floor(seg[:, 0]), jnp.floor(seg[:, 2])])
    hi = jnp.stack([jnp.ceil(seg[:, 1]), jnp.ceil(seg[:, 3])])
    lo = lo.astype(jnp.int32)
    hi = jnp.maximum(hi.astype(jnp.int32), lo + 1)
    w = hi - lo
    k = jnp.minimum(31 - lax.clz(w), LEVELS - 1)
    pw = jnp.left_shift(jnp.int32(1), k)
    hoff = (jnp.arange(2, dtype=jnp.int32) * (LEVELS * t))[:, None]
    idx1 = (hoff + k * t + lo).reshape(-1)
    idx2 = (hoff + k * t + (hi - pw)).reshape(-1)
    nrows = 2 * LEVELS * t
    idx1 = jnp.clip(idx1, 0, nrows - 1)
    idx2 = jnp.clip(idx2, 0, nrows - 1)

    d32 = b * ch // 2
    pyr = _build_pyramid(feature, b, t, ch)
    p2d = pyr.reshape(nrows, d32)

    idx = jnp.concatenate([idx1, idx2])
    g = _gather_rows(p2d, idx, 4 * nseg, d32)

    return _pair_max_transpose(g.reshape(2, 2, nseg, d32), b, ch, nseg)
